# baseline (speedup 1.0000x reference)
import functools

import jax
import jax.numpy as jnp
from jax import lax
from jax.experimental import pallas as pl
from jax.experimental.pallas import tpu as pltpu
from jax.experimental.pallas import tpu_sc as plsc

XW = 128
TW = 64
PKL = 4
FD = 64
MD = 16
H1 = 384
NW = 32
NC = 2
NS = 16


def _divisor_block(n, max_b, mult=8):
    best = None
    for b in range(mult, max_b + 1, mult):
        if n % b == 0:
            best = b
    if best is None:
        raise ValueError(f"no block for n={n} max={max_b}")
    return best


def _silu(x):
    return x * jax.nn.sigmoid(x)


def _ln(x, g, b, eps=1e-5):
    m = jnp.mean(x, axis=-1, keepdims=True)
    v = jnp.mean((x - m) * (x - m), axis=-1, keepdims=True)
    return (x - m) * jax.lax.rsqrt(v + eps) * g + b


def _pad_to(x, shape):
    pads = [(0, t - s) for s, t in zip(x.shape, shape)]
    return jnp.pad(x, pads)


def _table_row(c, h, n_rows):
    i32 = jnp.int32
    r1 = h[:, 0:32].astype(jnp.bfloat16).astype(jnp.float32)
    r2 = h[:, 32:64].astype(jnp.bfloat16).astype(jnp.float32)
    b1 = lax.bitcast_convert_type(r1, i32)
    b2 = lax.bitcast_convert_type(r2, i32)
    packed = jnp.bitwise_or(b1, lax.shift_right_logical(b2, 16))
    pf = lax.bitcast_convert_type(packed, jnp.float32)
    return jnp.concatenate(
        [c, jnp.zeros((n_rows, PKL - 3), jnp.float32), pf,
         jnp.zeros((n_rows, TW - PKL - 32), jnp.float32)], axis=1)


def _unpack_feats(pcol):
    i32 = jnp.int32
    pi = lax.bitcast_convert_type(pcol, i32)
    hi = lax.bitcast_convert_type(
        jnp.bitwise_and(pi, jnp.full(pi.shape, -65536, i32)), jnp.float32)
    lo = lax.bitcast_convert_type(
        lax.shift_left(pi, jnp.full(pi.shape, 16, i32)), jnp.float32)
    return jnp.concatenate([hi, lo], axis=1)


def _embed_body(atom, ring, hybr, arom, nfeat, Ta, Tr, Th, Tar, Wc, b0,
                W2, b2, W3, b3, xout, fout):
    def oh(ref, k):
        ids = ref[...]
        i = lax.broadcasted_iota(jnp.int32, (ids.shape[0], k), 1)
        return (i == ids).astype(jnp.float32)

    nf = nfeat[...]
    B = nf.shape[0]
    h = (jnp.dot(oh(atom, 16), Ta[...]) + jnp.dot(oh(ring, 8), Tr[...])
         + jnp.dot(oh(hybr, 8), Th[...]) + jnp.dot(oh(arom, 8), Tar[...])
         + jnp.dot(nf, Wc[...]) + b0[...])
    h = _silu(h)
    h = _silu(jnp.dot(h, W2[...]) + b2[...])
    h = _silu(jnp.dot(h, W3[...]) + b3[...])
    fout[...] = h
    xout[...] = _table_row(nf[:, 0:3], h, B)


def _edge_body(gr, W1, wdv, b1, W2, b2, g, b, out):
    grv = gr[...]
    B = grv.shape[0]
    d = grv[:, TW:TW + 3] - grv[:, 0:3]
    rd = jnp.sum(d * d, axis=1, keepdims=True)
    fd = _unpack_feats(grv[:, PKL:PKL + 32])
    fs = _unpack_feats(grv[:, TW + PKL:TW + PKL + 32])
    x = jnp.concatenate([fd, fs], axis=1).astype(jnp.bfloat16)
    t = (jnp.dot(x, W1[...], preferred_element_type=jnp.float32)
         + rd * wdv[...] + b1[...]).astype(jnp.bfloat16)
    t = _silu(t)
    m = _silu(jnp.dot(t, W2[...],
                      preferred_element_type=jnp.float32) + b2[...])
    out[...] = _ln(m, g[...], b[...])


def _make_node_body(nparts):
    def _node_body(xc, fc, *rest):
        parts = rest[:nparts]
        g2, b2, gn, bn, n1h, n1m, bn1, Wn2, bn2 = rest[nparts:nparts + 9]
        xout, fout = rest[nparts + 9:]
        x = xc[...]
        feats = fc[...]
        B = feats.shape[0]
        msum = parts[0][...]
        for pr in parts[1:]:
            msum = msum + pr[...]
        mi = _ln(msum, g2[...], b2[...])
        h = _ln(feats, gn[...], bn[...])
        u = _silu(jnp.dot(h, n1h[...]) + jnp.dot(mi, n1m[...]) + bn1[...])
        hnew = feats + jnp.dot(u, Wn2[...]) + bn2[...]
        fout[...] = hnew
        xout[...] = _table_row(x[:, 0:3], hnew, B)
    return _node_body


def _post_body(f1, f2, f3, P1, P2, P3, bp1, W2, bp2, W3, bp3, out):
    f = (jnp.dot(f1[...], P1[...]) + jnp.dot(f2[...], P2[...])
         + jnp.dot(f3[...], P3[...]) + bp1[...])
    f = _silu(f)
    f = _silu(jnp.dot(f, W2[...]) + bp2[...])
    out[...] = _silu(jnp.dot(f, W3[...]) + bp3[...])


def _final_body(p0, p1, lg, sl, rg, cl, Wp, TL, TS, TR, TCc, b1,
                W2, b2, W3, b3, W4, b4, out):
    def oh(ref, k):
        ids = ref[...]
        i = lax.broadcasted_iota(jnp.int32, (ids.shape[0], k), 1)
        return (i == ids).astype(jnp.float32)

    z = (jnp.dot(p0[...] + p1[...], Wp[...]) + jnp.dot(oh(lg, 16), TL[...])
         + jnp.dot(oh(sl, 16), TS[...]) + jnp.dot(oh(rg, 8), TR[...])
         + jnp.dot(oh(cl, 8), TCc[...]) + b1[...])
    z = _silu(z)
    z = _silu(jnp.dot(z, W2[...]) + b2[...])
    z = _silu(jnp.dot(z, W3[...]) + b3[...])
    out[...] = jnp.dot(z, W4[...]) + b4[...]


def _tc_call(body, grid, blocked, full, out_blocks, out_shapes):
    full = [a.reshape(1, -1) if a.ndim == 1 else a for a in full]
    in_specs = [pl.BlockSpec(bs, lambda i: (i, 0)) for _, bs in blocked]
    in_specs += [pl.BlockSpec(a.shape, lambda i, _r=len(a.shape): (0,) * _r)
                 for a in full]
    return pl.pallas_call(
        body,
        grid=(grid,),
        in_specs=in_specs,
        out_specs=[pl.BlockSpec(ob, lambda i: (i, 0)) for ob in out_blocks],
        out_shape=out_shapes,
    )(*[a for a, _ in blocked], *full)


def _sc_gather(xcat, sd_int, E):
    R = 2 * E
    PW = R // NW
    CH = _divisor_block(PW // 2, 256)
    NIT = PW // CH
    mesh = plsc.VectorSubcoreMesh(core_axis_name="c", subcore_axis_name="s")

    @functools.partial(
        pl.kernel, mesh=mesh,
        out_type=jax.ShapeDtypeStruct((R, TW), jnp.float32),
        scratch_types=[pltpu.VMEM((PW,), jnp.int32),
                       [pltpu.VMEM((CH, TW), jnp.float32) for _ in range(2)],
                       [pltpu.SemaphoreType.DMA for _ in range(2)],
                       [pltpu.SemaphoreType.DMA for _ in range(2)]],
        compiler_params=pltpu.CompilerParams(use_tc_tiling_on_sc=False),
    )
    def k(x_hbm, sd_hbm, g_hbm, idx_v, rows, gsem, wsem):
        wid = lax.axis_index("s") * NC + lax.axis_index("c")
        base = wid * PW
        pltpu.sync_copy(sd_hbm.at[pl.ds(base, PW)], idx_v)

        def gath(i, b):
            pltpu.async_copy(x_hbm.at[idx_v.at[pl.ds(i * CH, CH)]], rows[b],
                             gsem[b])

        gath(0, 0)
        gath(1, 1)

        def step(j, _):
            for b in (0, 1):
                i = 2 * j + b
                pltpu.make_async_copy(x_hbm.at[idx_v.at[pl.ds(0, CH)]],
                                      rows[b], gsem[b]).wait()
                pltpu.async_copy(rows[b],
                                 g_hbm.at[pl.ds(base + i * CH, CH)], wsem[b])

                @pl.when(i + 2 < NIT)
                def _n():
                    pltpu.make_async_copy(rows[b],
                                          g_hbm.at[pl.ds(base, CH)],
                                          wsem[b]).wait()
                    gath(i + 2, b)
            return _

        lax.fori_loop(0, NIT // 2, step, None)
        for b in (0, 1):
            pltpu.make_async_copy(rows[b], g_hbm.at[pl.ds(base, CH)],
                                  wsem[b]).wait()

    return k(xcat, sd_int)


def _sc_scatter(vals, idx, zeros, n_rows, width, max_ch):
    R = vals.shape[0]
    PW = R // NW
    CH = _divisor_block(PW, max_ch)
    NIT = PW // CH
    STR = n_rows // NS
    mesh = plsc.VectorSubcoreMesh(core_axis_name="c", subcore_axis_name="s")

    @functools.partial(
        pl.kernel, mesh=mesh,
        out_type=jax.ShapeDtypeStruct((NC, n_rows, width), jnp.float32),
        scratch_types=[pltpu.VMEM_SHARED((n_rows, width), jnp.float32),
                       pltpu.VMEM((CH,), jnp.int32),
                       pltpu.VMEM((CH, width), jnp.float32)],
        compiler_params=pltpu.CompilerParams(use_tc_tiling_on_sc=False),
    )
    def k(v_hbm, i_hbm, z_hbm, out_hbm, accum, idx_v, rows_v):
        c = lax.axis_index("c")
        s = lax.axis_index("s")
        wid = s * NC + c
        pltpu.sync_copy(z_hbm.at[pl.ds(s * STR, STR)],
                        accum.at[pl.ds(s * STR, STR)])
        plsc.subcore_barrier()
        base = wid * PW

        def step(i, _):
            off = base + i * CH
            pltpu.sync_copy(i_hbm.at[pl.ds(off, CH)], idx_v)
            pltpu.sync_copy(v_hbm.at[pl.ds(off, CH)], rows_v)
            pltpu.sync_copy(rows_v, accum.at[idx_v], add=True)
            return _

        lax.fori_loop(0, NIT, step, None)
        plsc.subcore_barrier()
        pltpu.sync_copy(accum.at[pl.ds(s * STR, STR)],
                        out_hbm.at[c, pl.ds(s * STR, STR)])

    return k(vals, idx, zeros)


def kernel(params, charges, crds_3d, atom_id, ring_id, hybr_id, arom_id,
           edge_index, batch, lgnd_id, slvn_id, rgnt_id, clst_id):
    N = charges.shape[0]
    E = edge_index.shape[1]
    G = lgnd_id.shape[0]
    NP = -(-N // (NW * 8)) * (NW * 8)
    EP = -(-E // (NW * 1600)) * (NW * 1600)
    NSPL = 2
    EH = EP // NSPL
    NSEG = -(-(G + 1) // 128) * 128
    BN = _divisor_block(NP, 2048)
    BE = _divisor_block(EH, 4096)
    f32 = jnp.float32
    bf16 = jnp.bfloat16

    p = params

    def padi(x):
        return jnp.pad(x.astype(jnp.int32), (0, NP - N)).reshape(NP, 1)

    nfeat = jnp.zeros((NP, 8), f32)
    nfeat = nfeat.at[:N, 0:3].set(crds_3d).at[:N, 3].set(charges[:, 0])
    atom_p, ring_p, hybr_p, arom_p = map(padi, (atom_id, ring_id, hybr_id, arom_id))
    src = jnp.pad(edge_index[0].astype(jnp.int32), (0, EP - E))
    dst = jnp.pad(edge_index[1].astype(jnp.int32), (0, EP - E),
                  constant_values=N)
    src_h = [src[h * EH:(h + 1) * EH] for h in range(NSPL)]
    dst_h = [dst[h * EH:(h + 1) * EH] for h in range(NSPL)]
    sd_h = [jnp.stack([dst_h[h], src_h[h]], axis=1).reshape(-1)
            for h in range(NSPL)]
    batch_p = jnp.pad(batch.astype(jnp.int32), (0, NP - N), constant_values=G)
    z16 = jnp.zeros((NP, MD), f32)
    z128 = jnp.zeros((NSEG, 2 * FD), f32)

    w1 = p["pre1"]["w"]
    Ta = _pad_to(p["atom_em"] @ w1[0:64], (16, 128))
    Tr = _pad_to(p["ring_em"] @ w1[64:128], (8, 128))
    Th = _pad_to(p["hybr_em"] @ w1[128:192], (8, 128))
    Tar = _pad_to(p["arom_em"] @ w1[192:256], (8, 128))
    cw = p["chrg"]["w"] @ w1[256:320]
    Wc = jnp.zeros((8, 128), f32).at[3].set(cw[0])
    b0 = p["pre1"]["b"] + p["chrg"]["b"] @ w1[256:320]

    xcat, feats = _tc_call(
        _embed_body, NP // BN,
        [(atom_p, (BN, 1)), (ring_p, (BN, 1)), (hybr_p, (BN, 1)),
         (arom_p, (BN, 1)), (nfeat, (BN, 8))],
        [Ta, Tr, Th, Tar, Wc, b0, p["pre2"]["w"], p["pre2"]["b"],
         p["pre3"]["w"], p["pre3"]["b"]],
        [(BN, TW), (BN, FD)],
        [jax.ShapeDtypeStruct((NP, TW), f32),
         jax.ShapeDtypeStruct((NP, FD), f32)])

    feats_list = []
    for kp in p["kernels"]:
        e1w, e1b = kp["e1"]["w"], kp["e1"]["b"]
        W1 = _pad_to(e1w[0:128], (XW, H1))
        wdv = _pad_to(e1w[128:129], (1, H1))
        b1 = _pad_to(e1b, (H1,))
        W2 = _pad_to(kp["e2"]["w"], (H1, MD))

        gath = [_sc_gather(xcat, sd_h[h], EH).reshape(EH, XW)
                for h in range(NSPL)]
        ms = [_tc_call(
            _edge_body, EH // BE,
            [(gath[h], (BE, XW))],
            [W1.astype(bf16), wdv, b1,
             W2.astype(bf16), kp["e2"]["b"], kp["en1_g"], kp["en1_b"]],
            [(BE, MD)], [jax.ShapeDtypeStruct((EH, MD), f32)])[0]
            for h in range(NSPL)]

        parts = [_sc_scatter(ms[h], dst_h[h], z16, NP, MD, 800)
                 for h in range(NSPL)]
        xcat, feats = _tc_call(
            _make_node_body(2 * NSPL), NP // BN,
            [(xcat, (BN, TW)), (feats, (BN, FD)),
             *[(parts[h][c], (BN, MD)) for h in range(NSPL) for c in (0, 1)]],
            [kp["en2_g"], kp["en2_b"], kp["nn1_g"], kp["nn1_b"],
             kp["n1"]["w"][0:FD], kp["n1"]["w"][FD:FD + MD], kp["n1"]["b"],
             kp["n2"]["w"], kp["n2"]["b"]],
            [(BN, TW), (BN, FD)],
            [jax.ShapeDtypeStruct((NP, TW), f32),
             jax.ShapeDtypeStruct((NP, FD), f32)])
        feats_list.append(feats)

    pw = p["post1"]["w"]
    f = _tc_call(
        _post_body, NP // BN,
        [(feats_list[0], (BN, FD)), (feats_list[1], (BN, FD)),
         (feats_list[2], (BN, FD))],
        [pw[0:64], pw[64:128], pw[128:192], p["post1"]["b"],
         p["post2"]["w"], p["post2"]["b"], p["post3"]["w"], p["post3"]["b"]],
        [(BN, 128)], [jax.ShapeDtypeStruct((NP, 128), f32)])[0]

    pooled = _sc_scatter(f, batch_p, z128, NSEG, 2 * FD, 784)

    wp1 = p["pp1"]["w"]
    TL = _pad_to(p["lig_emb"] @ wp1[128:192], (16, 512))
    TS = _pad_to(p["sol_emb"] @ wp1[192:256], (16, 512))
    TR = _pad_to(p["rgn_emb"] @ wp1[256:320], (8, 512))
    TCc = _pad_to(p["cat_emb"] @ wp1[320:384], (8, 512))

    def padg(x):
        return jnp.pad(x.astype(jnp.int32), (0, NSEG - G)).reshape(NSEG, 1)

    out = _tc_call(
        _final_body, 1,
        [(pooled[0], (NSEG, 128)), (pooled[1], (NSEG, 128)),
         (padg(lgnd_id), (NSEG, 1)), (padg(slvn_id), (NSEG, 1)),
         (padg(rgnt_id), (NSEG, 1)), (padg(clst_id), (NSEG, 1))],
        [wp1[0:128], TL, TS, TR, TCc, p["pp1"]["b"], p["pp2"]["w"],
         p["pp2"]["b"], p["pp3"]["w"], p["pp3"]["b"], p["pp4"]["w"],
         p["pp4"]["b"]],
        [(NSEG, 1)], [jax.ShapeDtypeStruct((NSEG, 1), f32)])[0]
    return out[:G, 0]


# speedup vs baseline: 1.0100x; 1.0100x over previous
import functools

import jax
import jax.numpy as jnp
from jax import lax
from jax.experimental import pallas as pl
from jax.experimental.pallas import tpu as pltpu
from jax.experimental.pallas import tpu_sc as plsc

XW = 128
TW = 64
PKL = 4
FD = 64
MD = 16
H1 = 384
NW = 32
NC = 2
NS = 16


def _divisor_block(n, max_b, mult=8):
    best = None
    for b in range(mult, max_b + 1, mult):
        if n % b == 0:
            best = b
    if best is None:
        raise ValueError(f"no block for n={n} max={max_b}")
    return best


def _silu(x):
    return x * jax.nn.sigmoid(x)


def _ln(x, g, b, eps=1e-5):
    m = jnp.mean(x, axis=-1, keepdims=True)
    v = jnp.mean((x - m) * (x - m), axis=-1, keepdims=True)
    return (x - m) * jax.lax.rsqrt(v + eps) * g + b


def _pad_to(x, shape):
    pads = [(0, t - s) for s, t in zip(x.shape, shape)]
    return jnp.pad(x, pads)


def _table_row(c, h, n_rows):
    i32 = jnp.int32
    r1 = h[:, 0:32].astype(jnp.bfloat16).astype(jnp.float32)
    r2 = h[:, 32:64].astype(jnp.bfloat16).astype(jnp.float32)
    b1 = lax.bitcast_convert_type(r1, i32)
    b2 = lax.bitcast_convert_type(r2, i32)
    packed = jnp.bitwise_or(b1, lax.shift_right_logical(b2, 16))
    pf = lax.bitcast_convert_type(packed, jnp.float32)
    return jnp.concatenate(
        [c, jnp.zeros((n_rows, PKL - 3), jnp.float32), pf,
         jnp.zeros((n_rows, TW - PKL - 32), jnp.float32)], axis=1)


def _unpack_feats(pcol):
    i32 = jnp.int32
    pi = lax.bitcast_convert_type(pcol, i32)
    hi = lax.bitcast_convert_type(
        jnp.bitwise_and(pi, jnp.full(pi.shape, -65536, i32)), jnp.float32)
    lo = lax.bitcast_convert_type(
        lax.shift_left(pi, jnp.full(pi.shape, 16, i32)), jnp.float32)
    return jnp.concatenate([hi, lo], axis=1)


def _embed_body(atom, ring, hybr, arom, nfeat, Ta, Tr, Th, Tar, Wc, b0,
                W2, b2, W3, b3, xout, fout):
    def oh(ref, k):
        ids = ref[...]
        i = lax.broadcasted_iota(jnp.int32, (ids.shape[0], k), 1)
        return (i == ids).astype(jnp.float32)

    nf = nfeat[...]
    B = nf.shape[0]
    h = (jnp.dot(oh(atom, 16), Ta[...]) + jnp.dot(oh(ring, 8), Tr[...])
         + jnp.dot(oh(hybr, 8), Th[...]) + jnp.dot(oh(arom, 8), Tar[...])
         + jnp.dot(nf, Wc[...]) + b0[...])
    h = _silu(h)
    h = _silu(jnp.dot(h, W2[...]) + b2[...])
    h = _silu(jnp.dot(h, W3[...]) + b3[...])
    fout[...] = h
    xout[...] = _table_row(nf[:, 0:3], h, B)


def _edge_body(gr, W1, wdv, b1, W2, b2, g, b, out):
    grv = gr[...]
    B = grv.shape[0]
    d = grv[:, TW:TW + 3] - grv[:, 0:3]
    rd = jnp.sum(d * d, axis=1, keepdims=True)
    fd = _unpack_feats(grv[:, PKL:PKL + 32])
    fs = _unpack_feats(grv[:, TW + PKL:TW + PKL + 32])
    x = jnp.concatenate([fd, fs], axis=1).astype(jnp.bfloat16)
    t = (jnp.dot(x, W1[...], preferred_element_type=jnp.float32)
         + rd * wdv[...] + b1[...])
    t = _silu(t)
    m = _silu(jnp.dot(t.astype(jnp.bfloat16), W2[...],
                      preferred_element_type=jnp.float32) + b2[...])
    out[...] = _ln(m, g[...], b[...])


def _make_node_body(nparts):
    def _node_body(xc, fc, *rest):
        parts = rest[:nparts]
        g2, b2, gn, bn, n1h, n1m, bn1, Wn2, bn2 = rest[nparts:nparts + 9]
        xout, fout = rest[nparts + 9:]
        x = xc[...]
        feats = fc[...]
        B = feats.shape[0]
        msum = parts[0][...]
        for pr in parts[1:]:
            msum = msum + pr[...]
        mi = _ln(msum, g2[...], b2[...])
        h = _ln(feats, gn[...], bn[...])
        u = _silu(jnp.dot(h, n1h[...]) + jnp.dot(mi, n1m[...]) + bn1[...])
        hnew = feats + jnp.dot(u, Wn2[...]) + bn2[...]
        fout[...] = hnew
        xout[...] = _table_row(x[:, 0:3], hnew, B)
    return _node_body


def _post_body(f1, f2, f3, P1, P2, P3, bp1, W2, bp2, W3, bp3, out):
    f = (jnp.dot(f1[...], P1[...]) + jnp.dot(f2[...], P2[...])
         + jnp.dot(f3[...], P3[...]) + bp1[...])
    f = _silu(f)
    f = _silu(jnp.dot(f, W2[...]) + bp2[...])
    out[...] = _silu(jnp.dot(f, W3[...]) + bp3[...])


def _final_body(p0, p1, lg, sl, rg, cl, Wp, TL, TS, TR, TCc, b1,
                W2, b2, W3, b3, W4, b4, out):
    def oh(ref, k):
        ids = ref[...]
        i = lax.broadcasted_iota(jnp.int32, (ids.shape[0], k), 1)
        return (i == ids).astype(jnp.float32)

    z = (jnp.dot(p0[...] + p1[...], Wp[...]) + jnp.dot(oh(lg, 16), TL[...])
         + jnp.dot(oh(sl, 16), TS[...]) + jnp.dot(oh(rg, 8), TR[...])
         + jnp.dot(oh(cl, 8), TCc[...]) + b1[...])
    z = _silu(z)
    z = _silu(jnp.dot(z, W2[...]) + b2[...])
    z = _silu(jnp.dot(z, W3[...]) + b3[...])
    out[...] = jnp.dot(z, W4[...]) + b4[...]


def _tc_call(body, grid, blocked, full, out_blocks, out_shapes):
    full = [a.reshape(1, -1) if a.ndim == 1 else a for a in full]
    in_specs = [pl.BlockSpec(bs, lambda i: (i, 0)) for _, bs in blocked]
    in_specs += [pl.BlockSpec(a.shape, lambda i, _r=len(a.shape): (0,) * _r)
                 for a in full]
    return pl.pallas_call(
        body,
        grid=(grid,),
        in_specs=in_specs,
        out_specs=[pl.BlockSpec(ob, lambda i: (i, 0)) for ob in out_blocks],
        out_shape=out_shapes,
    )(*[a for a, _ in blocked], *full)


def _sc_gather(xcat, sd_int, E):
    R = 2 * E
    PW = R // NW
    CH = _divisor_block(PW // 2, 256)
    NIT = PW // CH
    mesh = plsc.VectorSubcoreMesh(core_axis_name="c", subcore_axis_name="s")

    @functools.partial(
        pl.kernel, mesh=mesh,
        out_type=jax.ShapeDtypeStruct((R, TW), jnp.float32),
        scratch_types=[pltpu.VMEM((PW,), jnp.int32),
                       [pltpu.VMEM((CH, TW), jnp.float32) for _ in range(2)],
                       [pltpu.SemaphoreType.DMA for _ in range(2)],
                       [pltpu.SemaphoreType.DMA for _ in range(2)]],
        compiler_params=pltpu.CompilerParams(use_tc_tiling_on_sc=False),
    )
    def k(x_hbm, sd_hbm, g_hbm, idx_v, rows, gsem, wsem):
        wid = lax.axis_index("s") * NC + lax.axis_index("c")
        base = wid * PW
        pltpu.sync_copy(sd_hbm.at[pl.ds(base, PW)], idx_v)

        def gath(i, b):
            pltpu.async_copy(x_hbm.at[idx_v.at[pl.ds(i * CH, CH)]], rows[b],
                             gsem[b])

        gath(0, 0)
        gath(1, 1)

        def step(j, _):
            for b in (0, 1):
                i = 2 * j + b
                pltpu.make_async_copy(x_hbm.at[idx_v.at[pl.ds(0, CH)]],
                                      rows[b], gsem[b]).wait()
                pltpu.async_copy(rows[b],
                                 g_hbm.at[pl.ds(base + i * CH, CH)], wsem[b])

                @pl.when(i + 2 < NIT)
                def _n():
                    pltpu.make_async_copy(rows[b],
                                          g_hbm.at[pl.ds(base, CH)],
                                          wsem[b]).wait()
                    gath(i + 2, b)
            return _

        lax.fori_loop(0, NIT // 2, step, None)
        for b in (0, 1):
            pltpu.make_async_copy(rows[b], g_hbm.at[pl.ds(base, CH)],
                                  wsem[b]).wait()

    return k(xcat, sd_int)


def _sc_scatter(vals, idx, zeros, n_rows, width, max_ch):
    R = vals.shape[0]
    PW = R // NW
    CH = _divisor_block(PW, max_ch)
    NIT = PW // CH
    STR = n_rows // NS
    mesh = plsc.VectorSubcoreMesh(core_axis_name="c", subcore_axis_name="s")

    @functools.partial(
        pl.kernel, mesh=mesh,
        out_type=jax.ShapeDtypeStruct((NC, n_rows, width), jnp.float32),
        scratch_types=[pltpu.VMEM_SHARED((n_rows, width), jnp.float32),
                       pltpu.VMEM((CH,), jnp.int32),
                       pltpu.VMEM((CH, width), jnp.float32)],
        compiler_params=pltpu.CompilerParams(use_tc_tiling_on_sc=False),
    )
    def k(v_hbm, i_hbm, z_hbm, out_hbm, accum, idx_v, rows_v):
        c = lax.axis_index("c")
        s = lax.axis_index("s")
        wid = s * NC + c
        pltpu.sync_copy(z_hbm.at[pl.ds(s * STR, STR)],
                        accum.at[pl.ds(s * STR, STR)])
        plsc.subcore_barrier()
        base = wid * PW

        def step(i, _):
            off = base + i * CH
            pltpu.sync_copy(i_hbm.at[pl.ds(off, CH)], idx_v)
            pltpu.sync_copy(v_hbm.at[pl.ds(off, CH)], rows_v)
            pltpu.sync_copy(rows_v, accum.at[idx_v], add=True)
            return _

        lax.fori_loop(0, NIT, step, None)
        plsc.subcore_barrier()
        pltpu.sync_copy(accum.at[pl.ds(s * STR, STR)],
                        out_hbm.at[c, pl.ds(s * STR, STR)])

    return k(vals, idx, zeros)


def kernel(params, charges, crds_3d, atom_id, ring_id, hybr_id, arom_id,
           edge_index, batch, lgnd_id, slvn_id, rgnt_id, clst_id):
    N = charges.shape[0]
    E = edge_index.shape[1]
    G = lgnd_id.shape[0]
    NP = -(-N // (NW * 8)) * (NW * 8)
    EP = -(-E // (NW * 1600)) * (NW * 1600)
    NSPL = 2
    EH = EP // NSPL
    NSEG = -(-(G + 1) // 128) * 128
    BN = _divisor_block(NP, 2048)
    BE = _divisor_block(EH, 4096)
    f32 = jnp.float32
    bf16 = jnp.bfloat16

    p = params

    def padi(x):
        return jnp.pad(x.astype(jnp.int32), (0, NP - N)).reshape(NP, 1)

    nfeat = jnp.zeros((NP, 8), f32)
    nfeat = nfeat.at[:N, 0:3].set(crds_3d).at[:N, 3].set(charges[:, 0])
    atom_p, ring_p, hybr_p, arom_p = map(padi, (atom_id, ring_id, hybr_id, arom_id))
    src = jnp.pad(edge_index[0].astype(jnp.int32), (0, EP - E))
    dst = jnp.pad(edge_index[1].astype(jnp.int32), (0, EP - E),
                  constant_values=N)
    src_h = [src[h * EH:(h + 1) * EH] for h in range(NSPL)]
    dst_h = [dst[h * EH:(h + 1) * EH] for h in range(NSPL)]
    sd_h = [jnp.stack([dst_h[h], src_h[h]], axis=1).reshape(-1)
            for h in range(NSPL)]
    batch_p = jnp.pad(batch.astype(jnp.int32), (0, NP - N), constant_values=G)
    z16 = jnp.zeros((NP, MD), f32)
    z128 = jnp.zeros((NSEG, 2 * FD), f32)

    w1 = p["pre1"]["w"]
    Ta = _pad_to(p["atom_em"] @ w1[0:64], (16, 128))
    Tr = _pad_to(p["ring_em"] @ w1[64:128], (8, 128))
    Th = _pad_to(p["hybr_em"] @ w1[128:192], (8, 128))
    Tar = _pad_to(p["arom_em"] @ w1[192:256], (8, 128))
    cw = p["chrg"]["w"] @ w1[256:320]
    Wc = jnp.zeros((8, 128), f32).at[3].set(cw[0])
    b0 = p["pre1"]["b"] + p["chrg"]["b"] @ w1[256:320]

    xcat, feats = _tc_call(
        _embed_body, NP // BN,
        [(atom_p, (BN, 1)), (ring_p, (BN, 1)), (hybr_p, (BN, 1)),
         (arom_p, (BN, 1)), (nfeat, (BN, 8))],
        [Ta, Tr, Th, Tar, Wc, b0, p["pre2"]["w"], p["pre2"]["b"],
         p["pre3"]["w"], p["pre3"]["b"]],
        [(BN, TW), (BN, FD)],
        [jax.ShapeDtypeStruct((NP, TW), f32),
         jax.ShapeDtypeStruct((NP, FD), f32)])

    feats_list = []
    for kp in p["kernels"]:
        e1w, e1b = kp["e1"]["w"], kp["e1"]["b"]
        W1 = _pad_to(e1w[0:128], (XW, H1))
        wdv = _pad_to(e1w[128:129], (1, H1))
        b1 = _pad_to(e1b, (H1,))
        W2 = _pad_to(kp["e2"]["w"], (H1, MD))

        gath = [_sc_gather(xcat, sd_h[h], EH).reshape(EH, XW)
                for h in range(NSPL)]
        ms = [_tc_call(
            _edge_body, EH // BE,
            [(gath[h], (BE, XW))],
            [W1.astype(bf16), wdv, b1,
             W2.astype(bf16), kp["e2"]["b"], kp["en1_g"], kp["en1_b"]],
            [(BE, MD)], [jax.ShapeDtypeStruct((EH, MD), f32)])[0]
            for h in range(NSPL)]

        parts = [_sc_scatter(ms[h], dst_h[h], z16, NP, MD, 800)
                 for h in range(NSPL)]
        xcat, feats = _tc_call(
            _make_node_body(2 * NSPL), NP // BN,
            [(xcat, (BN, TW)), (feats, (BN, FD)),
             *[(parts[h][c], (BN, MD)) for h in range(NSPL) for c in (0, 1)]],
            [kp["en2_g"], kp["en2_b"], kp["nn1_g"], kp["nn1_b"],
             kp["n1"]["w"][0:FD], kp["n1"]["w"][FD:FD + MD], kp["n1"]["b"],
             kp["n2"]["w"], kp["n2"]["b"]],
            [(BN, TW), (BN, FD)],
            [jax.ShapeDtypeStruct((NP, TW), f32),
             jax.ShapeDtypeStruct((NP, FD), f32)])
        feats_list.append(feats)

    pw = p["post1"]["w"]
    f = _tc_call(
        _post_body, NP // BN,
        [(feats_list[0], (BN, FD)), (feats_list[1], (BN, FD)),
         (feats_list[2], (BN, FD))],
        [pw[0:64], pw[64:128], pw[128:192], p["post1"]["b"],
         p["post2"]["w"], p["post2"]["b"], p["post3"]["w"], p["post3"]["b"]],
        [(BN, 128)], [jax.ShapeDtypeStruct((NP, 128), f32)])[0]

    pooled = _sc_scatter(f, batch_p, z128, NSEG, 2 * FD, 784)

    wp1 = p["pp1"]["w"]
    TL = _pad_to(p["lig_emb"] @ wp1[128:192], (16, 512))
    TS = _pad_to(p["sol_emb"] @ wp1[192:256], (16, 512))
    TR = _pad_to(p["rgn_emb"] @ wp1[256:320], (8, 512))
    TCc = _pad_to(p["cat_emb"] @ wp1[320:384], (8, 512))

    def padg(x):
        return jnp.pad(x.astype(jnp.int32), (0, NSEG - G)).reshape(NSEG, 1)

    out = _tc_call(
        _final_body, 1,
        [(pooled[0], (NSEG, 128)), (pooled[1], (NSEG, 128)),
         (padg(lgnd_id), (NSEG, 1)), (padg(slvn_id), (NSEG, 1)),
         (padg(rgnt_id), (NSEG, 1)), (padg(clst_id), (NSEG, 1))],
        [wp1[0:128], TL, TS, TR, TCc, p["pp1"]["b"], p["pp2"]["w"],
         p["pp2"]["b"], p["pp3"]["w"], p["pp3"]["b"], p["pp4"]["w"],
         p["pp4"]["b"]],
        [(NSEG, 1)], [jax.ShapeDtypeStruct((NSEG, 1), f32)])[0]
    return out[:G, 0]
